# Initial kernel scaffold; baseline (speedup 1.0000x reference)
import jax
import jax.numpy as jnp
from jax.experimental import pallas as pl


def kernel(x, edge_index, W1, b1, W2, b2):
    raise NotImplementedError("write your pallas kernel here")


# speedup vs baseline: 1.9000x; 1.9000x over previous
import functools

import jax
import jax.numpy as jnp
from jax.experimental import pallas as pl

N = 10000
D_IN = 128
HID = 64
C = 40
K = 10
ALPHA = 0.1

ROW_BLK = 1000


def _mlp_body(x_ref, w1_ref, b1_ref, w2_ref, b2_ref, h_ref):
    h = jnp.maximum(x_ref[...] @ w1_ref[...] + b1_ref[...], 0.0)
    h_ref[...] = h @ w2_ref[...] + b2_ref[...]


def _mlp(x, W1, b1, W2, b2):
    grid = N // ROW_BLK
    return pl.pallas_call(
        _mlp_body,
        grid=(grid,),
        in_specs=[
            pl.BlockSpec((ROW_BLK, D_IN), lambda i: (i, 0)),
            pl.BlockSpec((D_IN, HID), lambda i: (0, 0)),
            pl.BlockSpec((1, HID), lambda i: (0, 0)),
            pl.BlockSpec((HID, C), lambda i: (0, 0)),
            pl.BlockSpec((1, C), lambda i: (0, 0)),
        ],
        out_specs=pl.BlockSpec((ROW_BLK, C), lambda i: (i, 0)),
        out_shape=jax.ShapeDtypeStruct((N, C), jnp.float32),
    )(x, W1, b1.reshape(1, HID), W2, b2.reshape(1, C))


def _lsm_body(x_ref, o_ref):
    x = x_ref[...]
    m = jnp.max(x, axis=1, keepdims=True)
    e = jnp.exp(x - m)
    s = jnp.sum(e, axis=1, keepdims=True)
    o_ref[...] = x - m - jnp.log(s)


def _log_softmax(z):
    grid = N // ROW_BLK
    return pl.pallas_call(
        _lsm_body,
        grid=(grid,),
        in_specs=[pl.BlockSpec((ROW_BLK, C), lambda i: (i, 0))],
        out_specs=pl.BlockSpec((ROW_BLK, C), lambda i: (i, 0)),
        out_shape=jax.ShapeDtypeStruct((N, C), jnp.float32),
    )(z)


def kernel(x, edge_index, W1, b1, W2, b2):
    h = _mlp(x, W1, b1, W2, b2)
    src = edge_index[0]
    dst = edge_index[1]
    deg = jnp.ones((N,), jnp.float32).at[dst].add(1.0)
    dinv = jax.lax.rsqrt(deg)
    u = h * dinv[:, None]
    out = h
    for _ in range(K):
        msg = u[src]
        agg = jnp.zeros_like(u).at[dst].add(msg)
        out = (1.0 - ALPHA) * (dinv[:, None] * (agg + u)) + ALPHA * h
        u = out * dinv[:, None]
    return _log_softmax(out)


# speedup vs baseline: 5.8508x; 3.0794x over previous
import functools

import jax
import jax.numpy as jnp
from jax import lax
from jax.experimental import pallas as pl
from jax.experimental.pallas import tpu as pltpu
from jax.experimental.pallas import tpu_sc as plsc

N = 10000
D_IN = 128
HID = 64
C = 40
K = 10
ALPHA = 0.1

NC, NS, L = 2, 16, 16
CP = 48
NPT = 320
NPAD = NC * NS * NPT
HALF = NPAD // 2
DUMMY = HALF
AGG_R = HALF + 16
ZPT = AGG_R // NS
E = 330000
E2 = E + 16
EB = 512
E3 = E2 + EB

_MESH = plsc.VectorSubcoreMesh(core_axis_name="c", subcore_axis_name="s")
_SC_PARAMS = pltpu.CompilerParams(
    use_tc_tiling_on_sc=False, needs_layout_passes=False)

ROW_BLK = 1024


def _mlp_body(x_ref, w1_ref, b1_ref, w2_ref, b2_ref, h_ref):
    h = jnp.maximum(x_ref[...] @ w1_ref[...] + b1_ref[...], 0.0)
    h_ref[...] = h @ w2_ref[...] + b2_ref[...]


def _mlp(x, W1, b1, W2, b2):
    blk = 1000
    return pl.pallas_call(
        _mlp_body,
        grid=(N // blk,),
        in_specs=[
            pl.BlockSpec((blk, D_IN), lambda i: (i, 0)),
            pl.BlockSpec((D_IN, HID), lambda i: (0, 0)),
            pl.BlockSpec((1, HID), lambda i: (0, 0)),
            pl.BlockSpec((HID, C), lambda i: (0, 0)),
            pl.BlockSpec((1, C), lambda i: (0, 0)),
        ],
        out_specs=pl.BlockSpec((blk, C), lambda i: (i, 0)),
        out_shape=jax.ShapeDtypeStruct((N, C), jnp.float32),
    )(x, W1, b1.reshape(1, HID), W2, b2.reshape(1, C))


def _lsm_body(u_ref, dinv_ref, o_ref):
    x = u_ref[...] / dinv_ref[...]
    col = lax.broadcasted_iota(jnp.int32, x.shape, 1)
    x = jnp.where(col < C, x, -jnp.inf)
    m = jnp.max(x, axis=1, keepdims=True)
    e = jnp.exp(x - m)
    s = jnp.sum(e, axis=1, keepdims=True)
    o_ref[...] = x - m - jnp.log(s)


def _log_softmax(u, dinv):
    return pl.pallas_call(
        _lsm_body,
        grid=(NPAD // ROW_BLK,),
        in_specs=[
            pl.BlockSpec((ROW_BLK, CP), lambda i: (i, 0)),
            pl.BlockSpec((ROW_BLK, 1), lambda i: (i, 0)),
        ],
        out_specs=pl.BlockSpec((ROW_BLK, CP), lambda i: (i, 0)),
        out_shape=jax.ShapeDtypeStruct((NPAD, CP), jnp.float32),
    )(u, dinv.reshape(NPAD, 1))


def _iota16():
    return lax.iota(jnp.int32, L)


def _lane_select(vec, lane):
    return jnp.sum(jnp.where(_iota16() == lane, vec, 0))


def _my_start_count(starts_hbm, counts_hbm, meta_v, c, s):
    pltpu.sync_copy(starts_hbm, meta_v.at[0])
    pltpu.sync_copy(counts_hbm, meta_v.at[1])
    srow = meta_v[0, c, pl.ds(0, NS)]
    crow = meta_v[1, c, pl.ds(0, NS)]
    start = pl.multiple_of(_lane_select(srow, s), 8)
    count = _lane_select(crow, s)
    return start, count


def _zero_agg(rows_v, agg_sp, s):
    @pl.loop(0, ZPT)
    def _(r):
        for j in range(CP // L):
            rows_v[r, pl.ds(j * L, L)] = jnp.zeros((L,), jnp.float32)

    pltpu.sync_copy(rows_v.at[pl.ds(0, ZPT)], agg_sp.at[pl.ds(s * ZPT, ZPT)])


def _load_and_mask(dst_hbm, dloc_v, base, end, c):
    for j in range(EB // 128):
        pltpu.sync_copy(dst_hbm.at[pl.ds(base + j * 128, 128)], dloc_v.at[j])
    for j in range(EB // 128):
        for l in range(128 // L):
            off = j * 128 + l * L
            d = dloc_v[j, pl.ds(l * L, L)]
            valid = _iota16() < (end - (base + off))
            dloc_v[j, pl.ds(l * L, L)] = jnp.where(valid, d - c * HALF, DUMMY)


def _deg_dinv(rows_v, dinv_v):
    zero16 = jnp.zeros((L,), jnp.int32)
    for g in range(NPT // L):
        rowidx = g * L + _iota16()
        deg = plsc.load_gather(rows_v, [rowidx, zero16])
        i = plsc.bitcast(deg, jnp.int32)
        i = jnp.int32(0x5F3759DF) - lax.shift_right_logical(i, 1)
        y = plsc.bitcast(i, jnp.float32)
        for _ in range(3):
            y = y * (1.5 - 0.5 * deg * y * y)
        dinv_v[pl.ds(g * L, L)] = y


def _prologue_kernel(dst_hbm, starts_hbm, counts_hbm, hp_hbm,
                     dinv_hbm, u0_hbm,
                     meta_v, dloc_v, ones_v, rows_v, hrow_v, dinv_v):
    c = lax.axis_index("c")
    s = lax.axis_index("s")
    start, count = _my_start_count(starts_hbm, counts_hbm, meta_v, c, s)
    end = start + count

    _zero_agg(rows_v, agg_sp, s)

    @pl.loop(0, 128)
    def _(r):
        for j in range(CP // L):
            ones_v[r, pl.ds(j * L, L)] = jnp.ones((L,), jnp.float32)

    plsc.subcore_barrier()

    nch = (count + EB - 1) // EB

    @pl.loop(0, nch)
    def _(i):
        base = pl.multiple_of(start + i * EB, 8)
        _load_and_mask(dst_hbm, dloc_v, base, end, c)
        for j in range(EB // 128):
            pltpu.sync_copy(ones_v, agg_sp.at[dloc_v.at[j]], add=True)

    plsc.subcore_barrier()

    g0 = pl.multiple_of(c * HALF + s * NPT, 8)
    pltpu.sync_copy(agg_sp.at[pl.ds(s * NPT, NPT)], rows_v.at[pl.ds(0, NPT)])
    _deg_dinv(rows_v, dinv_v)
    pltpu.sync_copy(dinv_v, dinv_hbm.at[pl.ds(g0, NPT)])
    pltpu.sync_copy(hp_hbm.at[pl.ds(g0, NPT)], hrow_v.at[pl.ds(0, NPT)])
    for g in range(NPT // L):
        rowidx = g * L + _iota16()
        dv = dinv_v[pl.ds(g * L, L)]
        for j in range(CP):
            colj = jnp.full((L,), j, jnp.int32)
            hval = plsc.load_gather(hrow_v, [rowidx, colj])
            plsc.store_scatter(hrow_v, [rowidx, colj], dv * hval)
    pltpu.sync_copy(hrow_v.at[pl.ds(0, NPT)], u0_hbm.at[pl.ds(g0, NPT)])


def _iter_kernel(u_hbm, hp_hbm, dinv_hbm, src_hbm, dst_hbm,
                 starts_hbm, counts_hbm, unew_hbm,
                 meta_v, srcA, srcB, dlocA, dlocB, rowsA, rowsB,
                 dinv_v, semA, semB):
    c = lax.axis_index("c")
    s = lax.axis_index("s")
    start, count = _my_start_count(starts_hbm, counts_hbm, meta_v, c, s)
    end = start + count

    _zero_agg(rowsA, agg_sp, s)
    plsc.subcore_barrier()

    nch = (count + EB - 1) // EB

    def issue(i, src_v, dloc_v, rows_v, sem):
        base = pl.multiple_of(start + i * EB, 8)
        for j in range(EB // 128):
            pltpu.sync_copy(src_hbm.at[pl.ds(base + j * 128, 128)],
                            src_v.at[j])
        _load_and_mask(dst_hbm, dloc_v, base, end, c)
        for j in range(EB // 128):
            pltpu.async_copy(u_hbm.at[src_v.at[j]],
                             rows_v.at[pl.ds(j * 128, 128)], sem)

    def drain_scatter(src_v, dloc_v, rows_v, sem):
        for j in range(EB // 128):
            pltpu.make_async_copy(u_hbm.at[src_v.at[j]],
                                  rows_v.at[pl.ds(j * 128, 128)], sem).wait()
        for j in range(EB // 128):
            pltpu.sync_copy(rows_v.at[pl.ds(j * 128, 128)],
                            agg_sp.at[dloc_v.at[j]], add=True)

    @pl.when(nch > 0)
    def _():
        issue(0, srcA, dlocA, rowsA, semA)

        @pl.loop(0, nch)
        def _(i):
            @pl.when(lax.rem(i, 2) == 0)
            def _():
                @pl.when(i + 1 < nch)
                def _():
                    issue(i + 1, srcB, dlocB, rowsB, semB)
                drain_scatter(srcA, dlocA, rowsA, semA)

            @pl.when(lax.rem(i, 2) == 1)
            def _():
                @pl.when(i + 1 < nch)
                def _():
                    issue(i + 1, srcA, dlocA, rowsA, semA)
                drain_scatter(srcB, dlocB, rowsB, semB)

    plsc.subcore_barrier()

    g0 = pl.multiple_of(c * HALF + s * NPT, 8)
    pltpu.sync_copy(agg_sp.at[pl.ds(s * NPT, NPT)], rowsA.at[pl.ds(0, NPT)])
    pltpu.sync_copy(hp_hbm.at[pl.ds(g0, NPT)], rowsB.at[pl.ds(0, NPT)])
    pltpu.sync_copy(dinv_hbm.at[pl.ds(g0, NPT)], dinv_v)
    for g in range(NPT // L):
        rowidx = g * L + _iota16()
        dv = dinv_v[pl.ds(g * L, L)]
        for j in range(CP):
            colj = jnp.full((L,), j, jnp.int32)
            a = plsc.load_gather(rowsA, [rowidx, colj])
            h = plsc.load_gather(rowsB, [rowidx, colj])
            out = (1.0 - ALPHA) * (dv * a) + ALPHA * h
            plsc.store_scatter(rowsA, [rowidx, colj], dv * out)
    pltpu.sync_copy(rowsA.at[pl.ds(0, NPT)], unew_hbm.at[pl.ds(g0, NPT)])


agg_sp = None


def _make_prologue():
    def body(dst_hbm, starts_hbm, counts_hbm, hp_hbm, dinv_hbm, u0_hbm,
             meta_v, dloc_v, ones_v, rows_v, hrow_v, dinv_v, agg):
        global agg_sp
        agg_sp = agg
        _prologue_kernel(dst_hbm, starts_hbm, counts_hbm, hp_hbm,
                         dinv_hbm, u0_hbm,
                         meta_v, dloc_v, ones_v, rows_v, hrow_v, dinv_v)

    return pl.kernel(
        body,
        out_type=[
            jax.ShapeDtypeStruct((NPAD,), jnp.float32),
            jax.ShapeDtypeStruct((NPAD, CP), jnp.float32),
        ],
        mesh=_MESH,
        compiler_params=_SC_PARAMS,
        scratch_types=[
            pltpu.VMEM((2, NC, NS), jnp.int32),
            pltpu.VMEM((EB // 128, 128), jnp.int32),
            pltpu.VMEM((128, CP), jnp.float32),
            pltpu.VMEM((ZPT + 7, CP), jnp.float32),
            pltpu.VMEM((NPT, CP), jnp.float32),
            pltpu.VMEM((NPT,), jnp.float32),
            pltpu.VMEM_SHARED((AGG_R, CP), jnp.float32),
        ],
    )


def _make_iter():
    def body(u_hbm, hp_hbm, dinv_hbm, src_hbm, dst_hbm, starts_hbm,
             counts_hbm, unew_hbm, meta_v, srcA, srcB, dlocA, dlocB,
             rowsA, rowsB, dinv_v, agg, semA, semB):
        global agg_sp
        agg_sp = agg
        _iter_kernel(u_hbm, hp_hbm, dinv_hbm, src_hbm, dst_hbm,
                     starts_hbm, counts_hbm, unew_hbm,
                     meta_v, srcA, srcB, dlocA, dlocB, rowsA, rowsB,
                     dinv_v, semA, semB)

    return pl.kernel(
        body,
        out_type=jax.ShapeDtypeStruct((NPAD, CP), jnp.float32),
        mesh=_MESH,
        compiler_params=_SC_PARAMS,
        scratch_types=[
            pltpu.VMEM((2, NC, NS), jnp.int32),
            pltpu.VMEM((EB // 128, 128), jnp.int32),
            pltpu.VMEM((EB // 128, 128), jnp.int32),
            pltpu.VMEM((EB // 128, 128), jnp.int32),
            pltpu.VMEM((EB // 128, 128), jnp.int32),
            pltpu.VMEM((EB, CP), jnp.float32),
            pltpu.VMEM((EB, CP), jnp.float32),
            pltpu.VMEM((NPT,), jnp.float32),
            pltpu.VMEM_SHARED((AGG_R, CP), jnp.float32),
            pltpu.SemaphoreType.DMA,
            pltpu.SemaphoreType.DMA,
        ],
    )


def kernel(x, edge_index, W1, b1, W2, b2):
    h = _mlp(x, W1, b1, W2, b2)
    hp = jnp.pad(h, ((0, NPAD - N), (0, CP - C)))

    loop_ids = jnp.arange(N, dtype=jnp.int32)
    src = jnp.concatenate([edge_index[0], loop_ids])
    dst = jnp.concatenate([edge_index[1], loop_ids])

    in0 = dst < HALF
    c0 = jnp.cumsum(in0.astype(jnp.int32))
    c1 = jnp.cumsum(jnp.logical_not(in0).astype(jnp.int32))
    n0 = c0[-1]
    n0p = (n0 + 7) // 8 * 8
    pos = jnp.where(in0, c0 - 1, n0p + c1 - 1)
    base_dst = jnp.where(jnp.arange(E3) < n0p, HALF, 2 * HALF
                         ).astype(jnp.int32)
    dst3 = base_dst.at[pos].set(dst)
    src3 = jnp.full((E3,), N, jnp.int32).at[pos].set(src)

    seg_start = jnp.stack([jnp.int32(0), n0p])
    seg_len = jnp.stack([n0p, jnp.int32(E2) - n0p])
    q = (seg_len + NS * 8 - 1) // (NS * 8) * 8
    s_idx = jnp.arange(NS, dtype=jnp.int32)
    seg_end = seg_start + seg_len
    starts = jnp.minimum(seg_start[:, None] + s_idx[None, :] * q[:, None],
                         seg_end[:, None]).astype(jnp.int32)
    counts = (jnp.minimum(starts + q[:, None], seg_end[:, None])
              - starts).astype(jnp.int32)

    dinv, u = _make_prologue()(dst3, starts, counts, hp)
    it = _make_iter()
    for _ in range(K):
        u = it(u, hp, dinv, src3, dst3, starts, counts)

    lsm = _log_softmax(u, dinv)
    return lsm[:N, :C]


# speedup vs baseline: 15.7680x; 2.6950x over previous
import jax
import jax.numpy as jnp
from jax import lax
from jax.experimental import pallas as pl
from jax.experimental.pallas import tpu as pltpu
from jax.experimental.pallas import tpu_sc as plsc

N = 10000
D_IN = 128
HID = 64
C = 40
K = 10
ALPHA = 0.1

NC, NS, L = 2, 16, 16
CP = 48
NPT = 320
NPAD = NC * NS * NPT
HALF = NPAD // 2
DUMMY = HALF
AGG_R = HALF + 16
ZPT = AGG_R // NS
E0 = 320000
E = E0 + N
EB_P = 512
NCH_P = (E + NS * EB_P - 1) // (NS * EB_P)
TQ = NCH_P * EB_P
EPTOT = NS * TQ
FLUSH = 4096
CBUF = FLUSH + EB_P + 16
RQ = 26112
RTOT = NC * NS * RQ
EB = 1024
JJ = EB // 128

_MESH = plsc.VectorSubcoreMesh(core_axis_name="c", subcore_axis_name="s")
_SC_PARAMS = pltpu.CompilerParams(
    use_tc_tiling_on_sc=False, needs_layout_passes=False)

ROW_BLK = 1024


def _mlp_body(x_ref, w1_ref, b1_ref, w2_ref, b2_ref, h_ref):
    h = jnp.maximum(x_ref[...] @ w1_ref[...] + b1_ref[...], 0.0)
    h_ref[...] = h @ w2_ref[...] + b2_ref[...]


def _mlp(x, W1, b1, W2, b2):
    blk = 1000
    return pl.pallas_call(
        _mlp_body,
        grid=(N // blk,),
        in_specs=[
            pl.BlockSpec((blk, D_IN), lambda i: (i, 0)),
            pl.BlockSpec((D_IN, HID), lambda i: (0, 0)),
            pl.BlockSpec((1, HID), lambda i: (0, 0)),
            pl.BlockSpec((HID, C), lambda i: (0, 0)),
            pl.BlockSpec((1, C), lambda i: (0, 0)),
        ],
        out_specs=pl.BlockSpec((blk, C), lambda i: (i, 0)),
        out_shape=jax.ShapeDtypeStruct((N, C), jnp.float32),
    )(x, W1, b1.reshape(1, HID), W2, b2.reshape(1, C))


def _lsm_body(u_ref, dinv_ref, o_ref):
    x = u_ref[...] / dinv_ref[...]
    col = lax.broadcasted_iota(jnp.int32, x.shape, 1)
    x = jnp.where(col < C, x, -jnp.inf)
    m = jnp.max(x, axis=1, keepdims=True)
    e = jnp.exp(x - m)
    s = jnp.sum(e, axis=1, keepdims=True)
    o_ref[...] = x - m - jnp.log(s)


def _log_softmax(u, dinv):
    return pl.pallas_call(
        _lsm_body,
        grid=(NPAD // ROW_BLK,),
        in_specs=[
            pl.BlockSpec((ROW_BLK, CP), lambda i: (i, 0)),
            pl.BlockSpec((ROW_BLK, 1), lambda i: (i, 0)),
        ],
        out_specs=pl.BlockSpec((ROW_BLK, CP), lambda i: (i, 0)),
        out_shape=jax.ShapeDtypeStruct((NPAD, CP), jnp.float32),
    )(u, dinv.reshape(NPAD, 1))


def _iota16():
    return lax.iota(jnp.int32, L)


def _lane_select(vec, lane):
    return jnp.sum(jnp.where(_iota16() == lane, vec, 0))


def _zero_agg(rows_v, agg, s):
    @pl.loop(0, ZPT)
    def _(r):
        for j in range(CP // L):
            rows_v[r, pl.ds(j * L, L)] = jnp.zeros((L,), jnp.float32)

    pltpu.sync_copy(rows_v.at[pl.ds(0, ZPT)], agg.at[pl.ds(s * ZPT, ZPT)])


def _deg_dinv(rows_v, dinv_v):
    zero16 = jnp.zeros((L,), jnp.int32)
    for g in range(NPT // L):
        rowidx = g * L + _iota16()
        deg = plsc.load_gather(rows_v, [rowidx, zero16])
        i = plsc.bitcast(deg, jnp.int32)
        i = jnp.int32(0x5F3759DF) - lax.shift_right_logical(i, 1)
        y = plsc.bitcast(i, jnp.float32)
        for _ in range(3):
            y = y * (1.5 - 0.5 * deg * y * y)
        dinv_v[pl.ds(g * L, L)] = y


def _prologue_kernel(src_hbm, dst_hbm, hp_hbm,
                     src2_hbm, dst2_hbm, counts_hbm, dinv_hbm, u0_hbm,
                     sbuf, dbuf, csrc, cdst, ones_v, rows_v, hrow_v,
                     dinv_v, cgrid_v, agg, counts_sp):
    c = lax.axis_index("c")
    s = lax.axis_index("s")

    _zero_agg(rows_v, agg, s)

    @pl.loop(0, 128)
    def _(r):
        for j in range(CP // L):
            ones_v[r, pl.ds(j * L, L)] = jnp.ones((L,), jnp.float32)

    @pl.loop(0, CBUF // L)
    def _(i):
        csrc[pl.ds(i * L, L)] = jnp.full((L,), N, jnp.int32)
        cdst[pl.ds(i * L, L)] = jnp.full((L,), DUMMY, jnp.int32)

    plsc.subcore_barrier()

    base0 = s * TQ
    regbase = (c * NS + s) * RQ

    def chunk(g, carry):
        cur, wcur = carry
        base = pl.multiple_of(base0 + g * EB_P, 8)
        for j in range(EB_P // 128):
            pltpu.sync_copy(src_hbm.at[pl.ds(base + j * 128, 128)],
                            sbuf.at[j])
            pltpu.sync_copy(dst_hbm.at[pl.ds(base + j * 128, 128)],
                            dbuf.at[j])
        for j in range(EB_P // 128):
            for l in range(128 // L):
                sv = sbuf[j, pl.ds(l * L, L)]
                dv = dbuf[j, pl.ds(l * L, L)]
                dloc = dv - c * HALF
                keep = jnp.logical_and(dloc >= 0, dloc < HALF)
                dd = jnp.where(keep, dloc, DUMMY)
                dbuf[j, pl.ds(l * L, L)] = dd
                plsc.store_compressed(csrc.at[pl.ds(cur, L)], sv, mask=keep)
                plsc.store_compressed(cdst.at[pl.ds(cur, L)], dd, mask=keep)
                cur = cur + jnp.sum(jnp.where(keep, 1, 0))
        for j in range(EB_P // 128):
            pltpu.sync_copy(ones_v, agg.at[dbuf.at[j]], add=True)

        @pl.when(cur >= FLUSH)
        def _():
            wc = pl.multiple_of(wcur, 8)
            pltpu.sync_copy(csrc.at[pl.ds(0, FLUSH)],
                            src2_hbm.at[pl.ds(regbase + wc, FLUSH)])
            pltpu.sync_copy(cdst.at[pl.ds(0, FLUSH)],
                            dst2_hbm.at[pl.ds(regbase + wc, FLUSH)])
            for t in range((CBUF - FLUSH) // L):
                csrc[pl.ds(t * L, L)] = csrc[pl.ds(FLUSH + t * L, L)]
                cdst[pl.ds(t * L, L)] = cdst[pl.ds(FLUSH + t * L, L)]

        flushed = jnp.where(cur >= FLUSH, 1, 0)
        return cur - flushed * FLUSH, wcur + flushed * FLUSH

    cur, wcur = pl.loop(0, NCH_P,
                        init_carry=(jnp.int32(0), jnp.int32(0)))(chunk)

    wc = pl.multiple_of(wcur, 8)
    pltpu.sync_copy(csrc.at[pl.ds(0, FLUSH + EB_P)],
                    src2_hbm.at[pl.ds(regbase + wc, FLUSH + EB_P)])
    pltpu.sync_copy(cdst.at[pl.ds(0, FLUSH + EB_P)],
                    dst2_hbm.at[pl.ds(regbase + wc, FLUSH + EB_P)])

    total = cur + wcur
    cgrid_v[0, pl.ds(0, L)] = jnp.where(_iota16() == s, total, 0)
    pltpu.sync_copy(cgrid_v.at[0], counts_sp.at[s])
    plsc.subcore_barrier()

    @pl.when(s == 0)
    def _():
        pltpu.sync_copy(counts_sp, cgrid_v)
        acc = jnp.zeros((L,), jnp.int32)
        for t in range(NS):
            acc = acc + cgrid_v[t, pl.ds(0, L)]
        cgrid_v[0, pl.ds(0, L)] = acc
        pltpu.sync_copy(cgrid_v.at[0], counts_hbm.at[c])

    g0 = pl.multiple_of(c * HALF + s * NPT, 8)
    pltpu.sync_copy(agg.at[pl.ds(s * NPT, NPT)], rows_v.at[pl.ds(0, NPT)])
    _deg_dinv(rows_v, dinv_v)
    pltpu.sync_copy(dinv_v, dinv_hbm.at[pl.ds(g0, NPT)])
    pltpu.sync_copy(hp_hbm.at[pl.ds(g0, NPT)], hrow_v.at[pl.ds(0, NPT)])
    for g in range(NPT // L):
        rowidx = g * L + _iota16()
        dv = dinv_v[pl.ds(g * L, L)]
        for j in range(CP):
            colj = jnp.full((L,), j, jnp.int32)
            hval = plsc.load_gather(hrow_v, [rowidx, colj])
            plsc.store_scatter(hrow_v, [rowidx, colj], dv * hval)
    pltpu.sync_copy(hrow_v.at[pl.ds(0, NPT)], u0_hbm.at[pl.ds(g0, NPT)])


def _iter_kernel(u_hbm, hp_hbm, dinv_hbm, src2_hbm, dst2_hbm, counts_hbm,
                 unew_hbm,
                 meta_v, srcA, srcB, dlocA, dlocB, rowsA, rowsB,
                 dinv_v, agg, gsemA, gsemB, ssemA, ssemB):
    c = lax.axis_index("c")
    s = lax.axis_index("s")
    pltpu.sync_copy(counts_hbm, meta_v)
    count = _lane_select(meta_v[c, pl.ds(0, NS)], s)
    start = pl.multiple_of((c * NS + s) * RQ, 8)
    end = start + count

    _zero_agg(rowsA, agg, s)
    plsc.subcore_barrier()

    nch = (count + EB - 1) // EB

    def load_idx(i, src_v, dloc_v):
        base = pl.multiple_of(start + i * EB, 8)
        for j in range(JJ):
            pltpu.sync_copy(src2_hbm.at[pl.ds(base + j * 128, 128)],
                            src_v.at[j])
            pltpu.sync_copy(dst2_hbm.at[pl.ds(base + j * 128, 128)],
                            dloc_v.at[j])
        for j in range(JJ):
            for l in range(128 // L):
                off = j * 128 + l * L
                d = dloc_v[j, pl.ds(l * L, L)]
                valid = _iota16() < (end - (base + off))
                dloc_v[j, pl.ds(l * L, L)] = jnp.where(valid, d, DUMMY)

    def fire_gather(src_v, rows_v, sem):
        for j in range(JJ):
            pltpu.async_copy(u_hbm.at[src_v.at[j]],
                             rows_v.at[pl.ds(j * 128, 128)], sem)

    def wait_gather(src_v, rows_v, sem):
        for j in range(JJ):
            pltpu.make_async_copy(u_hbm.at[src_v.at[j]],
                                  rows_v.at[pl.ds(j * 128, 128)], sem).wait()

    def fire_scatter(dloc_v, rows_v, sem):
        for j in range(JJ):
            pltpu.async_copy(rows_v.at[pl.ds(j * 128, 128)],
                             agg.at[dloc_v.at[j]], sem, add=True)

    def wait_scatter(dloc_v, rows_v, sem):
        for j in range(JJ):
            pltpu.make_async_copy(rows_v.at[pl.ds(j * 128, 128)],
                                  agg.at[dloc_v.at[j]], sem).wait()

    @pl.when(nch > 0)
    def _():
        load_idx(0, srcA, dlocA)
        fire_gather(srcA, rowsA, gsemA)

        @pl.loop(0, nch)
        def _(i):
            @pl.when(lax.rem(i, 2) == 0)
            def _():
                @pl.when(i + 1 < nch)
                def _():
                    load_idx(i + 1, srcB, dlocB)

                    @pl.when(i >= 1)
                    def _():
                        wait_scatter(dlocB, rowsB, ssemB)

                    fire_gather(srcB, rowsB, gsemB)

                wait_gather(srcA, rowsA, gsemA)
                fire_scatter(dlocA, rowsA, ssemA)

            @pl.when(lax.rem(i, 2) == 1)
            def _():
                @pl.when(i + 1 < nch)
                def _():
                    load_idx(i + 1, srcA, dlocA)
                    wait_scatter(dlocA, rowsA, ssemA)
                    fire_gather(srcA, rowsA, gsemA)

                wait_gather(srcB, rowsB, gsemB)
                fire_scatter(dlocB, rowsB, ssemB)

        @pl.when(lax.rem(nch, 2) == 1)
        def _():
            wait_scatter(dlocA, rowsA, ssemA)

            @pl.when(nch > 1)
            def _():
                wait_scatter(dlocB, rowsB, ssemB)

        @pl.when(lax.rem(nch, 2) == 0)
        def _():
            wait_scatter(dlocB, rowsB, ssemB)

            @pl.when(nch > 1)
            def _():
                wait_scatter(dlocA, rowsA, ssemA)

    plsc.subcore_barrier()

    g0 = pl.multiple_of(c * HALF + s * NPT, 8)
    pltpu.sync_copy(agg.at[pl.ds(s * NPT, NPT)], rowsA.at[pl.ds(0, NPT)])
    pltpu.sync_copy(hp_hbm.at[pl.ds(g0, NPT)], rowsB.at[pl.ds(0, NPT)])
    pltpu.sync_copy(dinv_hbm.at[pl.ds(g0, NPT)], dinv_v)
    for g in range(NPT // L):
        rowidx = g * L + _iota16()
        dv = dinv_v[pl.ds(g * L, L)]
        for j in range(CP):
            colj = jnp.full((L,), j, jnp.int32)
            a = plsc.load_gather(rowsA, [rowidx, colj])
            h = plsc.load_gather(rowsB, [rowidx, colj])
            out = (1.0 - ALPHA) * (dv * a) + ALPHA * h
            plsc.store_scatter(rowsA, [rowidx, colj], dv * out)
    pltpu.sync_copy(rowsA.at[pl.ds(0, NPT)], unew_hbm.at[pl.ds(g0, NPT)])


_prologue = pl.kernel(
    _prologue_kernel,
    out_type=[
        jax.ShapeDtypeStruct((RTOT,), jnp.int32),
        jax.ShapeDtypeStruct((RTOT,), jnp.int32),
        jax.ShapeDtypeStruct((NC, NS), jnp.int32),
        jax.ShapeDtypeStruct((NPAD,), jnp.float32),
        jax.ShapeDtypeStruct((NPAD, CP), jnp.float32),
    ],
    mesh=_MESH,
    compiler_params=_SC_PARAMS,
    scratch_types=[
        pltpu.VMEM((EB_P // 128, 128), jnp.int32),
        pltpu.VMEM((EB_P // 128, 128), jnp.int32),
        pltpu.VMEM((CBUF,), jnp.int32),
        pltpu.VMEM((CBUF,), jnp.int32),
        pltpu.VMEM((128, CP), jnp.float32),
        pltpu.VMEM((ZPT + 7, CP), jnp.float32),
        pltpu.VMEM((NPT, CP), jnp.float32),
        pltpu.VMEM((NPT,), jnp.float32),
        pltpu.VMEM((NS, L), jnp.int32),
        pltpu.VMEM_SHARED((AGG_R, CP), jnp.float32),
        pltpu.VMEM_SHARED((NS, L), jnp.int32),
    ],
)

_iterk = pl.kernel(
    _iter_kernel,
    out_type=jax.ShapeDtypeStruct((NPAD, CP), jnp.float32),
    mesh=_MESH,
    compiler_params=_SC_PARAMS,
    scratch_types=[
        pltpu.VMEM((NC, NS), jnp.int32),
        pltpu.VMEM((JJ, 128), jnp.int32),
        pltpu.VMEM((JJ, 128), jnp.int32),
        pltpu.VMEM((JJ, 128), jnp.int32),
        pltpu.VMEM((JJ, 128), jnp.int32),
        pltpu.VMEM((EB, CP), jnp.float32),
        pltpu.VMEM((EB, CP), jnp.float32),
        pltpu.VMEM((NPT,), jnp.float32),
        pltpu.VMEM_SHARED((AGG_R, CP), jnp.float32),
        pltpu.SemaphoreType.DMA,
        pltpu.SemaphoreType.DMA,
        pltpu.SemaphoreType.DMA,
        pltpu.SemaphoreType.DMA,
    ],
)


def kernel(x, edge_index, W1, b1, W2, b2):
    h = _mlp(x, W1, b1, W2, b2)
    hp = jnp.pad(h, ((0, NPAD - N), (0, CP - C)))

    loop_ids = jnp.arange(N, dtype=jnp.int32)
    srcp = jnp.concatenate([
        edge_index[0], loop_ids,
        jnp.full((EPTOT - E,), N, jnp.int32)])
    dstp = jnp.concatenate([
        edge_index[1], loop_ids,
        jnp.full((EPTOT - E,), 2 * HALF + 1, jnp.int32)])

    src2, dst2, counts, dinv, u = _prologue(srcp, dstp, hp)
    for _ in range(K):
        u = _iterk(u, hp, dinv, src2, dst2, counts)

    lsm = _log_softmax(u, dinv)
    return lsm[:N, :C]


# speedup vs baseline: 21.9729x; 1.3935x over previous
import jax
import jax.numpy as jnp
from jax import lax
from jax.experimental import pallas as pl
from jax.experimental.pallas import tpu as pltpu
from jax.experimental.pallas import tpu_sc as plsc

N = 10000
D_IN = 128
HID = 64
C = 40
K = 10
ALPHA = 0.1

NC, NS, L = 2, 16, 16
CP = 48
NPT = 320
NPAD = NC * NS * NPT
HALF = NPAD // 2
DUMMY = HALF
AGG_R = HALF + 16
ZPT = AGG_R // NS
E0 = 320000
E = E0 + N
EB_P = 512
NCH_P = (E + NS * EB_P - 1) // (NS * EB_P)
TQ = NCH_P * EB_P
EPTOT = NS * TQ
FLUSH = 4096
CBUF = FLUSH + EB_P + 16
RQ = 26112
RTOT = NC * NS * RQ
EB = 1024
JJ = EB // 128

_MESH = plsc.VectorSubcoreMesh(core_axis_name="c", subcore_axis_name="s")
_SC_PARAMS = pltpu.CompilerParams(
    use_tc_tiling_on_sc=False, needs_layout_passes=False)

ROW_BLK = 1024


def _mlp_body(x_ref, w1_ref, b1_ref, w2_ref, b2_ref, h_ref):
    h = jnp.maximum(x_ref[...] @ w1_ref[...] + b1_ref[...], 0.0)
    h_ref[...] = h @ w2_ref[...] + b2_ref[...]


def _mlp(x, W1, b1, W2, b2):
    blk = 1000
    return pl.pallas_call(
        _mlp_body,
        grid=(N // blk,),
        in_specs=[
            pl.BlockSpec((blk, D_IN), lambda i: (i, 0)),
            pl.BlockSpec((D_IN, HID), lambda i: (0, 0)),
            pl.BlockSpec((1, HID), lambda i: (0, 0)),
            pl.BlockSpec((HID, C), lambda i: (0, 0)),
            pl.BlockSpec((1, C), lambda i: (0, 0)),
        ],
        out_specs=pl.BlockSpec((blk, C), lambda i: (i, 0)),
        out_shape=jax.ShapeDtypeStruct((N, C), jnp.float32),
    )(x, W1, b1.reshape(1, HID), W2, b2.reshape(1, C))


def _lsm_body(u_ref, dinv_ref, o_ref):
    x = u_ref[...] / dinv_ref[...]
    col = lax.broadcasted_iota(jnp.int32, x.shape, 1)
    x = jnp.where(col < C, x, -jnp.inf)
    m = jnp.max(x, axis=1, keepdims=True)
    e = jnp.exp(x - m)
    s = jnp.sum(e, axis=1, keepdims=True)
    o_ref[...] = x - m - jnp.log(s)


def _log_softmax(u, dinv):
    return pl.pallas_call(
        _lsm_body,
        grid=(NPAD // ROW_BLK,),
        in_specs=[
            pl.BlockSpec((ROW_BLK, CP), lambda i: (i, 0)),
            pl.BlockSpec((ROW_BLK, 1), lambda i: (i, 0)),
        ],
        out_specs=pl.BlockSpec((ROW_BLK, CP), lambda i: (i, 0)),
        out_shape=jax.ShapeDtypeStruct((NPAD, CP), jnp.float32),
    )(u, dinv.reshape(NPAD, 1))


def _iota16():
    return lax.iota(jnp.int32, L)


def _lane_select(vec, lane):
    return jnp.sum(jnp.where(_iota16() == lane, vec, 0))


def _zero_agg(rows_v, agg, s):
    @pl.loop(0, ZPT)
    def _(r):
        for j in range(CP // L):
            rows_v[r, pl.ds(j * L, L)] = jnp.zeros((L,), jnp.float32)

    pltpu.sync_copy(rows_v.at[pl.ds(0, ZPT)], agg.at[pl.ds(s * ZPT, ZPT)])


def _deg_dinv(rows_v, dinv_v):
    zero16 = jnp.zeros((L,), jnp.int32)
    for g in range(NPT // L):
        rowidx = g * L + _iota16()
        deg = plsc.load_gather(rows_v, [rowidx, zero16])
        i = plsc.bitcast(deg, jnp.int32)
        i = jnp.int32(0x5F3759DF) - lax.shift_right_logical(i, 1)
        y = plsc.bitcast(i, jnp.float32)
        for _ in range(3):
            y = y * (1.5 - 0.5 * deg * y * y)
        dinv_v[pl.ds(g * L, L)] = y


def _prologue_kernel(src_hbm, dst_hbm, hp_hbm,
                     src2_hbm, dst2_hbm, counts_hbm, dinv_hbm, u0_hbm,
                     sbuf, dbuf, csrc, cdst, ones_v, rows_v, hrow_v,
                     dinv_v, cgrid_v, agg, counts_sp, isem):
    c = lax.axis_index("c")
    s = lax.axis_index("s")

    _zero_agg(rows_v, agg, s)

    @pl.loop(0, 128)
    def _(r):
        for j in range(CP // L):
            ones_v[r, pl.ds(j * L, L)] = jnp.ones((L,), jnp.float32)

    @pl.loop(0, CBUF // L)
    def _(i):
        csrc[pl.ds(i * L, L)] = jnp.full((L,), N, jnp.int32)
        cdst[pl.ds(i * L, L)] = jnp.full((L,), DUMMY, jnp.int32)

    plsc.subcore_barrier()

    base0 = s * TQ
    regbase = (c * NS + s) * RQ

    def chunk(g, carry):
        cur, wcur = carry
        base = pl.multiple_of(base0 + g * EB_P, 8)
        for j in range(EB_P // 128):
            pltpu.async_copy(src_hbm.at[pl.ds(base + j * 128, 128)],
                             sbuf.at[j], isem)
            pltpu.async_copy(dst_hbm.at[pl.ds(base + j * 128, 128)],
                             dbuf.at[j], isem)
        for j in range(EB_P // 128):
            pltpu.make_async_copy(src_hbm.at[pl.ds(base + j * 128, 128)],
                                  sbuf.at[j], isem).wait()
            pltpu.make_async_copy(dst_hbm.at[pl.ds(base + j * 128, 128)],
                                  dbuf.at[j], isem).wait()
        for j in range(EB_P // 128):
            for l in range(128 // L):
                sv = sbuf[j, pl.ds(l * L, L)]
                dv = dbuf[j, pl.ds(l * L, L)]
                dloc = dv - c * HALF
                keep = jnp.logical_and(dloc >= 0, dloc < HALF)
                dd = jnp.where(keep, dloc, DUMMY)
                dbuf[j, pl.ds(l * L, L)] = dd
                plsc.store_compressed(csrc.at[pl.ds(cur, L)], sv, mask=keep)
                plsc.store_compressed(cdst.at[pl.ds(cur, L)], dd, mask=keep)
                cur = cur + jnp.sum(jnp.where(keep, 1, 0))
        for j in range(EB_P // 128):
            pltpu.sync_copy(ones_v, agg.at[dbuf.at[j]], add=True)

        @pl.when(cur >= FLUSH)
        def _():
            wc = pl.multiple_of(wcur, 8)
            pltpu.sync_copy(csrc.at[pl.ds(0, FLUSH)],
                            src2_hbm.at[pl.ds(regbase + wc, FLUSH)])
            pltpu.sync_copy(cdst.at[pl.ds(0, FLUSH)],
                            dst2_hbm.at[pl.ds(regbase + wc, FLUSH)])
            for t in range((CBUF - FLUSH) // L):
                csrc[pl.ds(t * L, L)] = csrc[pl.ds(FLUSH + t * L, L)]
                cdst[pl.ds(t * L, L)] = cdst[pl.ds(FLUSH + t * L, L)]

        flushed = jnp.where(cur >= FLUSH, 1, 0)
        return cur - flushed * FLUSH, wcur + flushed * FLUSH

    cur, wcur = pl.loop(0, NCH_P,
                        init_carry=(jnp.int32(0), jnp.int32(0)))(chunk)

    wc = pl.multiple_of(wcur, 8)
    pltpu.sync_copy(csrc.at[pl.ds(0, FLUSH + EB_P)],
                    src2_hbm.at[pl.ds(regbase + wc, FLUSH + EB_P)])
    pltpu.sync_copy(cdst.at[pl.ds(0, FLUSH + EB_P)],
                    dst2_hbm.at[pl.ds(regbase + wc, FLUSH + EB_P)])

    total = cur + wcur
    cgrid_v[0, pl.ds(0, L)] = jnp.where(_iota16() == s, total, 0)
    pltpu.sync_copy(cgrid_v.at[0], counts_sp.at[s])
    plsc.subcore_barrier()

    @pl.when(s == 0)
    def _():
        pltpu.sync_copy(counts_sp, cgrid_v)
        acc = jnp.zeros((L,), jnp.int32)
        for t in range(NS):
            acc = acc + cgrid_v[t, pl.ds(0, L)]
        cgrid_v[0, pl.ds(0, L)] = acc
        pltpu.sync_copy(cgrid_v.at[0], counts_hbm.at[c])

    g0 = pl.multiple_of(c * HALF + s * NPT, 8)
    pltpu.sync_copy(agg.at[pl.ds(s * NPT, NPT)], rows_v.at[pl.ds(0, NPT)])
    _deg_dinv(rows_v, dinv_v)
    pltpu.sync_copy(dinv_v, dinv_hbm.at[pl.ds(g0, NPT)])
    pltpu.sync_copy(hp_hbm.at[pl.ds(g0, NPT)], hrow_v.at[pl.ds(0, NPT)])
    for g in range(NPT // L):
        rowidx = g * L + _iota16()
        dv = dinv_v[pl.ds(g * L, L)]
        for j in range(CP):
            colj = jnp.full((L,), j, jnp.int32)
            hval = plsc.load_gather(hrow_v, [rowidx, colj])
            plsc.store_scatter(hrow_v, [rowidx, colj], dv * hval)
    pltpu.sync_copy(hrow_v.at[pl.ds(0, NPT)], u0_hbm.at[pl.ds(g0, NPT)])


def _iter_kernel(u_hbm, hp_hbm, dinv_hbm, src2_hbm, dst2_hbm, counts_hbm,
                 unew_hbm,
                 meta_v, srcA, srcB, dlocA, dlocB, rowsA, rowsB,
                 dinv_v, agg, gsemA, gsemB, ssemA, ssemB, isem):
    c = lax.axis_index("c")
    s = lax.axis_index("s")
    pltpu.sync_copy(counts_hbm, meta_v)
    count = _lane_select(meta_v[c, pl.ds(0, NS)], s)
    start = pl.multiple_of((c * NS + s) * RQ, 8)
    end = start + count

    _zero_agg(rowsA, agg, s)
    plsc.subcore_barrier()

    nch = (count + EB - 1) // EB

    def load_idx(i, src_v, dloc_v):
        base = pl.multiple_of(start + i * EB, 8)
        for j in range(JJ):
            pltpu.async_copy(src2_hbm.at[pl.ds(base + j * 128, 128)],
                             src_v.at[j], isem)
            pltpu.async_copy(dst2_hbm.at[pl.ds(base + j * 128, 128)],
                             dloc_v.at[j], isem)
        for j in range(JJ):
            pltpu.make_async_copy(src2_hbm.at[pl.ds(base + j * 128, 128)],
                                  src_v.at[j], isem).wait()
            pltpu.make_async_copy(dst2_hbm.at[pl.ds(base + j * 128, 128)],
                                  dloc_v.at[j], isem).wait()
        for j in range(JJ):
            for l in range(128 // L):
                off = j * 128 + l * L
                d = dloc_v[j, pl.ds(l * L, L)]
                valid = _iota16() < (end - (base + off))
                dloc_v[j, pl.ds(l * L, L)] = jnp.where(valid, d, DUMMY)

    def fire_gather(src_v, rows_v, sem):
        for j in range(JJ):
            pltpu.async_copy(u_hbm.at[src_v.at[j]],
                             rows_v.at[pl.ds(j * 128, 128)], sem)

    def wait_gather(src_v, rows_v, sem):
        for j in range(JJ):
            pltpu.make_async_copy(u_hbm.at[src_v.at[j]],
                                  rows_v.at[pl.ds(j * 128, 128)], sem).wait()

    def fire_scatter(dloc_v, rows_v, sem):
        for j in range(JJ):
            pltpu.async_copy(rows_v.at[pl.ds(j * 128, 128)],
                             agg.at[dloc_v.at[j]], sem, add=True)

    def wait_scatter(dloc_v, rows_v, sem):
        for j in range(JJ):
            pltpu.make_async_copy(rows_v.at[pl.ds(j * 128, 128)],
                                  agg.at[dloc_v.at[j]], sem).wait()

    @pl.when(nch > 0)
    def _():
        load_idx(0, srcA, dlocA)
        fire_gather(srcA, rowsA, gsemA)

        @pl.loop(0, nch)
        def _(i):
            @pl.when(lax.rem(i, 2) == 0)
            def _():
                @pl.when(i + 1 < nch)
                def _():
                    load_idx(i + 1, srcB, dlocB)

                    @pl.when(i >= 1)
                    def _():
                        wait_scatter(dlocB, rowsB, ssemB)

                    fire_gather(srcB, rowsB, gsemB)

                wait_gather(srcA, rowsA, gsemA)
                fire_scatter(dlocA, rowsA, ssemA)

            @pl.when(lax.rem(i, 2) == 1)
            def _():
                @pl.when(i + 1 < nch)
                def _():
                    load_idx(i + 1, srcA, dlocA)
                    wait_scatter(dlocA, rowsA, ssemA)
                    fire_gather(srcA, rowsA, gsemA)

                wait_gather(srcB, rowsB, gsemB)
                fire_scatter(dlocB, rowsB, ssemB)

        @pl.when(lax.rem(nch, 2) == 1)
        def _():
            wait_scatter(dlocA, rowsA, ssemA)

            @pl.when(nch > 1)
            def _():
                wait_scatter(dlocB, rowsB, ssemB)

        @pl.when(lax.rem(nch, 2) == 0)
        def _():
            wait_scatter(dlocB, rowsB, ssemB)

            @pl.when(nch > 1)
            def _():
                wait_scatter(dlocA, rowsA, ssemA)

    plsc.subcore_barrier()

    g0 = pl.multiple_of(c * HALF + s * NPT, 8)
    pltpu.sync_copy(agg.at[pl.ds(s * NPT, NPT)], rowsA.at[pl.ds(0, NPT)])
    pltpu.sync_copy(hp_hbm.at[pl.ds(g0, NPT)], rowsB.at[pl.ds(0, NPT)])
    pltpu.sync_copy(dinv_hbm.at[pl.ds(g0, NPT)], dinv_v)
    for g in range(NPT // L):
        rowidx = g * L + _iota16()
        dv = dinv_v[pl.ds(g * L, L)]
        for j in range(CP):
            colj = jnp.full((L,), j, jnp.int32)
            a = plsc.load_gather(rowsA, [rowidx, colj])
            h = plsc.load_gather(rowsB, [rowidx, colj])
            out = (1.0 - ALPHA) * (dv * a) + ALPHA * h
            plsc.store_scatter(rowsA, [rowidx, colj], dv * out)
    pltpu.sync_copy(rowsA.at[pl.ds(0, NPT)], unew_hbm.at[pl.ds(g0, NPT)])


_prologue = pl.kernel(
    _prologue_kernel,
    out_type=[
        jax.ShapeDtypeStruct((RTOT,), jnp.int32),
        jax.ShapeDtypeStruct((RTOT,), jnp.int32),
        jax.ShapeDtypeStruct((NC, NS), jnp.int32),
        jax.ShapeDtypeStruct((NPAD,), jnp.float32),
        jax.ShapeDtypeStruct((NPAD, CP), jnp.float32),
    ],
    mesh=_MESH,
    compiler_params=_SC_PARAMS,
    scratch_types=[
        pltpu.VMEM((EB_P // 128, 128), jnp.int32),
        pltpu.VMEM((EB_P // 128, 128), jnp.int32),
        pltpu.VMEM((CBUF,), jnp.int32),
        pltpu.VMEM((CBUF,), jnp.int32),
        pltpu.VMEM((128, CP), jnp.float32),
        pltpu.VMEM((ZPT + 7, CP), jnp.float32),
        pltpu.VMEM((NPT, CP), jnp.float32),
        pltpu.VMEM((NPT,), jnp.float32),
        pltpu.VMEM((NS, L), jnp.int32),
        pltpu.VMEM_SHARED((AGG_R, CP), jnp.float32),
        pltpu.VMEM_SHARED((NS, L), jnp.int32),
        pltpu.SemaphoreType.DMA,
    ],
)

_iterk = pl.kernel(
    _iter_kernel,
    out_type=jax.ShapeDtypeStruct((NPAD, CP), jnp.float32),
    mesh=_MESH,
    compiler_params=_SC_PARAMS,
    scratch_types=[
        pltpu.VMEM((NC, NS), jnp.int32),
        pltpu.VMEM((JJ, 128), jnp.int32),
        pltpu.VMEM((JJ, 128), jnp.int32),
        pltpu.VMEM((JJ, 128), jnp.int32),
        pltpu.VMEM((JJ, 128), jnp.int32),
        pltpu.VMEM((EB, CP), jnp.float32),
        pltpu.VMEM((EB, CP), jnp.float32),
        pltpu.VMEM((NPT,), jnp.float32),
        pltpu.VMEM_SHARED((AGG_R, CP), jnp.float32),
        pltpu.SemaphoreType.DMA,
        pltpu.SemaphoreType.DMA,
        pltpu.SemaphoreType.DMA,
        pltpu.SemaphoreType.DMA,
        pltpu.SemaphoreType.DMA,
    ],
)


def kernel(x, edge_index, W1, b1, W2, b2):
    h = _mlp(x, W1, b1, W2, b2)
    hp = jnp.pad(h, ((0, NPAD - N), (0, CP - C)))

    loop_ids = jnp.arange(N, dtype=jnp.int32)
    srcp = jnp.concatenate([
        edge_index[0], loop_ids,
        jnp.full((EPTOT - E,), N, jnp.int32)])
    dstp = jnp.concatenate([
        edge_index[1], loop_ids,
        jnp.full((EPTOT - E,), 2 * HALF + 1, jnp.int32)])

    src2, dst2, counts, dinv, u = _prologue(srcp, dstp, hp)
    for _ in range(K):
        u = _iterk(u, hp, dinv, src2, dst2, counts)

    lsm = _log_softmax(u, dinv)
    return lsm[:N, :C]


# speedup vs baseline: 22.7077x; 1.0334x over previous
import jax
import jax.numpy as jnp
from jax import lax
from jax.experimental import pallas as pl
from jax.experimental.pallas import tpu as pltpu
from jax.experimental.pallas import tpu_sc as plsc

N = 10000
D_IN = 128
HID = 64
C = 40
K = 10
ALPHA = 0.1

NC, NS, L = 2, 16, 16
CP = 48
NPT = 320
NPAD = NC * NS * NPT
HALF = NPAD // 2
DUMMY = HALF
AGG_R = HALF + 16
ZPT = AGG_R // NS
E0 = 320000
E = E0 + N
EB_P = 512
NCH_P = (E + NS * EB_P - 1) // (NS * EB_P)
TQ = NCH_P * EB_P
EPTOT = NS * TQ
FLUSH = 4096
CBUF = FLUSH + EB_P + 16
RQ = 26112
RTOT = NC * NS * RQ
EB = 1024
JJ = EB // 128

_MESH = plsc.VectorSubcoreMesh(core_axis_name="c", subcore_axis_name="s")
_SC_PARAMS = pltpu.CompilerParams(
    use_tc_tiling_on_sc=False, needs_layout_passes=False)

ROW_BLK = 1024


def _mlp_body(x_ref, w1_ref, b1_ref, w2_ref, b2_ref, h_ref):
    h = jnp.maximum(x_ref[...] @ w1_ref[...] + b1_ref[...], 0.0)
    h_ref[...] = h @ w2_ref[...] + b2_ref[...]


def _mlp(x, W1, b1, W2, b2):
    blk = 1000
    return pl.pallas_call(
        _mlp_body,
        grid=(N // blk,),
        in_specs=[
            pl.BlockSpec((blk, D_IN), lambda i: (i, 0)),
            pl.BlockSpec((D_IN, HID), lambda i: (0, 0)),
            pl.BlockSpec((1, HID), lambda i: (0, 0)),
            pl.BlockSpec((HID, C), lambda i: (0, 0)),
            pl.BlockSpec((1, C), lambda i: (0, 0)),
        ],
        out_specs=pl.BlockSpec((blk, C), lambda i: (i, 0)),
        out_shape=jax.ShapeDtypeStruct((N, C), jnp.float32),
    )(x, W1, b1.reshape(1, HID), W2, b2.reshape(1, C))


def _lsm_body(u_ref, dinv_ref, o_ref):
    x = u_ref[...] / dinv_ref[...]
    col = lax.broadcasted_iota(jnp.int32, x.shape, 1)
    x = jnp.where(col < C, x, -jnp.inf)
    m = jnp.max(x, axis=1, keepdims=True)
    e = jnp.exp(x - m)
    s = jnp.sum(e, axis=1, keepdims=True)
    o_ref[...] = x - m - jnp.log(s)


def _log_softmax(u, dinv):
    return pl.pallas_call(
        _lsm_body,
        grid=(NPAD // ROW_BLK,),
        in_specs=[
            pl.BlockSpec((ROW_BLK, CP), lambda i: (i, 0)),
            pl.BlockSpec((ROW_BLK, 1), lambda i: (i, 0)),
        ],
        out_specs=pl.BlockSpec((ROW_BLK, CP), lambda i: (i, 0)),
        out_shape=jax.ShapeDtypeStruct((NPAD, CP), jnp.float32),
    )(u, dinv.reshape(NPAD, 1))


def _iota16():
    return lax.iota(jnp.int32, L)


def _lane_select(vec, lane):
    return jnp.sum(jnp.where(_iota16() == lane, vec, 0))


def _zero_agg(rows_v, agg, s):
    @pl.loop(0, ZPT)
    def _(r):
        for j in range(CP // L):
            rows_v[r, pl.ds(j * L, L)] = jnp.zeros((L,), jnp.float32)

    pltpu.sync_copy(rows_v.at[pl.ds(0, ZPT)], agg.at[pl.ds(s * ZPT, ZPT)])


def _deg_dinv(rows_v, dinv_v):
    zero16 = jnp.zeros((L,), jnp.int32)
    for g in range(NPT // L):
        rowidx = g * L + _iota16()
        deg = plsc.load_gather(rows_v, [rowidx, zero16])
        i = plsc.bitcast(deg, jnp.int32)
        i = jnp.int32(0x5F3759DF) - lax.shift_right_logical(i, 1)
        y = plsc.bitcast(i, jnp.float32)
        for _ in range(3):
            y = y * (1.5 - 0.5 * deg * y * y)
        dinv_v[pl.ds(g * L, L)] = y


def _prologue_kernel(src_hbm, dst_hbm, hp_hbm,
                     src2_hbm, dst2_hbm, counts_hbm, dinv_hbm, u0_hbm,
                     sbuf, dbuf, csrc, cdst, ones_v, rows_v, hrow_v,
                     dinv_v, cgrid_v, agg, counts_sp, isem):
    c = lax.axis_index("c")
    s = lax.axis_index("s")

    _zero_agg(rows_v, agg, s)

    @pl.loop(0, 128)
    def _(r):
        for j in range(CP // L):
            ones_v[r, pl.ds(j * L, L)] = jnp.ones((L,), jnp.float32)

    @pl.loop(0, CBUF // L)
    def _(i):
        csrc[pl.ds(i * L, L)] = jnp.full((L,), N, jnp.int32)
        cdst[pl.ds(i * L, L)] = jnp.full((L,), DUMMY, jnp.int32)

    plsc.subcore_barrier()

    base0 = s * TQ
    regbase = (c * NS + s) * RQ

    def chunk(g, carry):
        cur, wcur = carry
        base = pl.multiple_of(base0 + g * EB_P, 8)
        for j in range(EB_P // 128):
            pltpu.async_copy(src_hbm.at[pl.ds(base + j * 128, 128)],
                             sbuf.at[j], isem)
            pltpu.async_copy(dst_hbm.at[pl.ds(base + j * 128, 128)],
                             dbuf.at[j], isem)
        for j in range(EB_P // 128):
            pltpu.make_async_copy(src_hbm.at[pl.ds(base + j * 128, 128)],
                                  sbuf.at[j], isem).wait()
            pltpu.make_async_copy(dst_hbm.at[pl.ds(base + j * 128, 128)],
                                  dbuf.at[j], isem).wait()
        for j in range(EB_P // 128):
            for l in range(128 // L):
                sv = sbuf[j, pl.ds(l * L, L)]
                dv = dbuf[j, pl.ds(l * L, L)]
                dloc = dv - c * HALF
                keep = jnp.logical_and(dloc >= 0, dloc < HALF)
                dd = jnp.where(keep, dloc, DUMMY)
                dbuf[j, pl.ds(l * L, L)] = dd
                plsc.store_compressed(csrc.at[pl.ds(cur, L)], sv, mask=keep)
                plsc.store_compressed(cdst.at[pl.ds(cur, L)], dd, mask=keep)
                cur = cur + jnp.sum(jnp.where(keep, 1, 0))
        for j in range(EB_P // 128):
            pltpu.sync_copy(ones_v, agg.at[dbuf.at[j]], add=True)

        @pl.when(cur >= FLUSH)
        def _():
            wc = pl.multiple_of(wcur, 8)
            pltpu.sync_copy(csrc.at[pl.ds(0, FLUSH)],
                            src2_hbm.at[pl.ds(regbase + wc, FLUSH)])
            pltpu.sync_copy(cdst.at[pl.ds(0, FLUSH)],
                            dst2_hbm.at[pl.ds(regbase + wc, FLUSH)])
            for t in range((CBUF - FLUSH) // L):
                csrc[pl.ds(t * L, L)] = csrc[pl.ds(FLUSH + t * L, L)]
                cdst[pl.ds(t * L, L)] = cdst[pl.ds(FLUSH + t * L, L)]

        flushed = jnp.where(cur >= FLUSH, 1, 0)
        return cur - flushed * FLUSH, wcur + flushed * FLUSH

    cur, wcur = pl.loop(0, NCH_P,
                        init_carry=(jnp.int32(0), jnp.int32(0)))(chunk)

    wc = pl.multiple_of(wcur, 8)
    pltpu.sync_copy(csrc.at[pl.ds(0, FLUSH + EB_P)],
                    src2_hbm.at[pl.ds(regbase + wc, FLUSH + EB_P)])
    pltpu.sync_copy(cdst.at[pl.ds(0, FLUSH + EB_P)],
                    dst2_hbm.at[pl.ds(regbase + wc, FLUSH + EB_P)])

    total = cur + wcur
    cgrid_v[0, pl.ds(0, L)] = jnp.where(_iota16() == s, total, 0)
    pltpu.sync_copy(cgrid_v.at[0], counts_sp.at[s])
    plsc.subcore_barrier()

    @pl.when(s == 0)
    def _():
        pltpu.sync_copy(counts_sp, cgrid_v)
        acc = jnp.zeros((L,), jnp.int32)
        for t in range(NS):
            acc = acc + cgrid_v[t, pl.ds(0, L)]
        cgrid_v[0, pl.ds(0, L)] = acc
        pltpu.sync_copy(cgrid_v.at[0], counts_hbm.at[c])

    g0 = pl.multiple_of(c * HALF + s * NPT, 8)
    pltpu.sync_copy(agg.at[pl.ds(s * NPT, NPT)], rows_v.at[pl.ds(0, NPT)])
    _deg_dinv(rows_v, dinv_v)
    pltpu.sync_copy(dinv_v, dinv_hbm.at[pl.ds(g0, NPT)])
    pltpu.sync_copy(hp_hbm.at[pl.ds(g0, NPT)], hrow_v.at[pl.ds(0, NPT)])
    for g in range(NPT // L):
        rowidx = g * L + _iota16()
        dv = dinv_v[pl.ds(g * L, L)]
        for j in range(CP):
            colj = jnp.full((L,), j, jnp.int32)
            hval = plsc.load_gather(hrow_v, [rowidx, colj])
            plsc.store_scatter(hrow_v, [rowidx, colj], dv * hval)
    pltpu.sync_copy(hrow_v.at[pl.ds(0, NPT)], u0_hbm.at[pl.ds(g0, NPT)])


def _iter_kernel(u_hbm, hp_hbm, dinv_hbm, src2_hbm, dst2_hbm, counts_hbm,
                 unew_hbm,
                 meta_v, srcA, srcB, dlocA, dlocB, rowsA, rowsB,
                 dinv_v, agg, gsemA, gsemB, ssemA, ssemB, isem):
    c = lax.axis_index("c")
    s = lax.axis_index("s")
    pltpu.sync_copy(counts_hbm, meta_v)
    count = _lane_select(meta_v[c, pl.ds(0, NS)], s)
    start = pl.multiple_of((c * NS + s) * RQ, 8)
    end = start + count

    _zero_agg(rowsA, agg, s)
    plsc.subcore_barrier()

    nch = (count + EB - 1) // EB

    def load_idx(i, src_v, dloc_v):
        base = pl.multiple_of(start + i * EB, 8)
        for j in range(JJ):
            pltpu.async_copy(src2_hbm.at[pl.ds(base + j * 128, 128)],
                             src_v.at[j], isem)
            pltpu.async_copy(dst2_hbm.at[pl.ds(base + j * 128, 128)],
                             dloc_v.at[j], isem)
        for j in range(JJ):
            pltpu.make_async_copy(src2_hbm.at[pl.ds(base + j * 128, 128)],
                                  src_v.at[j], isem).wait()
            pltpu.make_async_copy(dst2_hbm.at[pl.ds(base + j * 128, 128)],
                                  dloc_v.at[j], isem).wait()
        for j in range(JJ):
            for l in range(128 // L):
                off = j * 128 + l * L
                d = dloc_v[j, pl.ds(l * L, L)]
                valid = _iota16() < (end - (base + off))
                dloc_v[j, pl.ds(l * L, L)] = jnp.where(valid, d, DUMMY)

    def fire_gather(src_v, rows_v, sem):
        for j in range(JJ):
            pltpu.async_copy(u_hbm.at[src_v.at[j]],
                             rows_v.at[pl.ds(j * 128, 128)], sem)

    def wait_gather(src_v, rows_v, sem):
        for j in range(JJ):
            pltpu.make_async_copy(u_hbm.at[src_v.at[j]],
                                  rows_v.at[pl.ds(j * 128, 128)], sem).wait()

    def fire_scatter(dloc_v, rows_v, sem):
        for j in range(JJ):
            pltpu.async_copy(rows_v.at[pl.ds(j * 128, 128)],
                             agg.at[dloc_v.at[j]], sem, add=True)

    def wait_scatter(dloc_v, rows_v, sem):
        for j in range(JJ):
            pltpu.make_async_copy(rows_v.at[pl.ds(j * 128, 128)],
                                  agg.at[dloc_v.at[j]], sem).wait()

    @pl.when(nch > 0)
    def _():
        load_idx(0, srcA, dlocA)
        fire_gather(srcA, rowsA, gsemA)

        @pl.loop(0, nch)
        def _(i):
            @pl.when(lax.rem(i, 2) == 0)
            def _():
                @pl.when(i + 1 < nch)
                def _():
                    @pl.when(i >= 1)
                    def _():
                        wait_scatter(dlocB, rowsB, ssemB)

                    load_idx(i + 1, srcB, dlocB)
                    fire_gather(srcB, rowsB, gsemB)

                wait_gather(srcA, rowsA, gsemA)
                fire_scatter(dlocA, rowsA, ssemA)

            @pl.when(lax.rem(i, 2) == 1)
            def _():
                @pl.when(i + 1 < nch)
                def _():
                    wait_scatter(dlocA, rowsA, ssemA)
                    load_idx(i + 1, srcA, dlocA)
                    fire_gather(srcA, rowsA, gsemA)

                wait_gather(srcB, rowsB, gsemB)
                fire_scatter(dlocB, rowsB, ssemB)

        @pl.when(lax.rem(nch, 2) == 1)
        def _():
            wait_scatter(dlocA, rowsA, ssemA)

            @pl.when(nch > 1)
            def _():
                wait_scatter(dlocB, rowsB, ssemB)

        @pl.when(lax.rem(nch, 2) == 0)
        def _():
            wait_scatter(dlocB, rowsB, ssemB)

            @pl.when(nch > 1)
            def _():
                wait_scatter(dlocA, rowsA, ssemA)

    plsc.subcore_barrier()

    g0 = pl.multiple_of(c * HALF + s * NPT, 8)
    pltpu.sync_copy(agg.at[pl.ds(s * NPT, NPT)], rowsA.at[pl.ds(0, NPT)])
    pltpu.sync_copy(hp_hbm.at[pl.ds(g0, NPT)], rowsB.at[pl.ds(0, NPT)])
    pltpu.sync_copy(dinv_hbm.at[pl.ds(g0, NPT)], dinv_v)
    for g in range(NPT // L):
        rowidx = g * L + _iota16()
        dv = dinv_v[pl.ds(g * L, L)]
        for j in range(CP):
            colj = jnp.full((L,), j, jnp.int32)
            a = plsc.load_gather(rowsA, [rowidx, colj])
            h = plsc.load_gather(rowsB, [rowidx, colj])
            out = (1.0 - ALPHA) * (dv * a) + ALPHA * h
            plsc.store_scatter(rowsA, [rowidx, colj], dv * out)
    pltpu.sync_copy(rowsA.at[pl.ds(0, NPT)], unew_hbm.at[pl.ds(g0, NPT)])


_prologue = pl.kernel(
    _prologue_kernel,
    out_type=[
        jax.ShapeDtypeStruct((RTOT,), jnp.int32),
        jax.ShapeDtypeStruct((RTOT,), jnp.int32),
        jax.ShapeDtypeStruct((NC, NS), jnp.int32),
        jax.ShapeDtypeStruct((NPAD,), jnp.float32),
        jax.ShapeDtypeStruct((NPAD, CP), jnp.float32),
    ],
    mesh=_MESH,
    compiler_params=_SC_PARAMS,
    scratch_types=[
        pltpu.VMEM((EB_P // 128, 128), jnp.int32),
        pltpu.VMEM((EB_P // 128, 128), jnp.int32),
        pltpu.VMEM((CBUF,), jnp.int32),
        pltpu.VMEM((CBUF,), jnp.int32),
        pltpu.VMEM((128, CP), jnp.float32),
        pltpu.VMEM((ZPT + 7, CP), jnp.float32),
        pltpu.VMEM((NPT, CP), jnp.float32),
        pltpu.VMEM((NPT,), jnp.float32),
        pltpu.VMEM((NS, L), jnp.int32),
        pltpu.VMEM_SHARED((AGG_R, CP), jnp.float32),
        pltpu.VMEM_SHARED((NS, L), jnp.int32),
        pltpu.SemaphoreType.DMA,
    ],
)

_iterk = pl.kernel(
    _iter_kernel,
    out_type=jax.ShapeDtypeStruct((NPAD, CP), jnp.float32),
    mesh=_MESH,
    compiler_params=_SC_PARAMS,
    scratch_types=[
        pltpu.VMEM((NC, NS), jnp.int32),
        pltpu.VMEM((JJ, 128), jnp.int32),
        pltpu.VMEM((JJ, 128), jnp.int32),
        pltpu.VMEM((JJ, 128), jnp.int32),
        pltpu.VMEM((JJ, 128), jnp.int32),
        pltpu.VMEM((EB, CP), jnp.float32),
        pltpu.VMEM((EB, CP), jnp.float32),
        pltpu.VMEM((NPT,), jnp.float32),
        pltpu.VMEM_SHARED((AGG_R, CP), jnp.float32),
        pltpu.SemaphoreType.DMA,
        pltpu.SemaphoreType.DMA,
        pltpu.SemaphoreType.DMA,
        pltpu.SemaphoreType.DMA,
        pltpu.SemaphoreType.DMA,
    ],
)


def kernel(x, edge_index, W1, b1, W2, b2):
    h = _mlp(x, W1, b1, W2, b2)
    hp = jnp.pad(h, ((0, NPAD - N), (0, CP - C)))

    loop_ids = jnp.arange(N, dtype=jnp.int32)
    srcp = jnp.concatenate([
        edge_index[0], loop_ids,
        jnp.full((EPTOT - E,), N, jnp.int32)])
    dstp = jnp.concatenate([
        edge_index[1], loop_ids,
        jnp.full((EPTOT - E,), 2 * HALF + 1, jnp.int32)])

    src2, dst2, counts, dinv, u = _prologue(srcp, dstp, hp)
    for _ in range(K):
        u = _iterk(u, hp, dinv, src2, dst2, counts)

    lsm = _log_softmax(u, dinv)
    return lsm[:N, :C]
